# Initial kernel scaffold; baseline (speedup 1.0000x reference)
import jax
import jax.numpy as jnp
from jax.experimental import pallas as pl


def kernel(x, edge_index, edge_type, Wq, Wk, Wv, R, W_o, ln_gamma, ln_beta):
    raise NotImplementedError("write your pallas kernel here")


# speedup vs baseline: 1.4832x; 1.4832x over previous
import functools
import math

import jax
import jax.numpy as jnp
from jax import lax
from jax.experimental import pallas as pl
from jax.experimental.pallas import tpu as pltpu
from jax.experimental.pallas import tpu_sc as plsc

N = 10000
E = 320000
D = 128
H = 4
HD = D // H
NR = 64

NC = 2
NS = 16
NW = NC * NS
L = 16

N_PAD = 10240
RW = N_PAD // NW

EW = E // NW
GR2 = 200
NCH2 = EW // GR2

CH = 4000
NCH4 = E // CH
GR4 = 128
MCAP = 4096

BE = 3200
BN = 2048

_mesh = plsc.VectorSubcoreMesh(core_axis_name="c", subcore_axis_name="s")


def _wid():
    return lax.axis_index("s") * NC + lax.axis_index("c")


def _q_body(x_ref, wq_ref, q_ref):
    q_ref[...] = jnp.dot(x_ref[...], wq_ref[...],
                         preferred_element_type=jnp.float32)


def _q_proj(x, Wq):
    return pl.pallas_call(
        _q_body,
        out_shape=jax.ShapeDtypeStruct((N, D), jnp.float32),
    )(x, Wq)


@functools.partial(
    pl.kernel,
    out_type=(jax.ShapeDtypeStruct((E, D), jnp.float32),
              jax.ShapeDtypeStruct((E, D), jnp.float32)),
    mesh=_mesh,
    scratch_types=[
        pltpu.VMEM((GR2,), jnp.int32),
        pltpu.VMEM((GR2, D), jnp.float32),
        pltpu.VMEM((GR2, D), jnp.float32),
        pltpu.VMEM((GR2, D), jnp.float32),
        pltpu.SemaphoreType.DMA,
    ],
)
def _sc_gather(x_hbm, q_hbm, r_hbm, src_hbm, dst_hbm, et_hbm,
               msg_hbm, qd_hbm, idx_v, xs_v, rel_v, qd_v, sem):
    ebase = _wid() * EW

    def chunk(c, carry):
        off = ebase + c * GR2
        pltpu.sync_copy(src_hbm.at[pl.ds(off, GR2)], idx_v)
        pltpu.async_copy(x_hbm.at[idx_v], xs_v, sem).wait()
        pltpu.sync_copy(et_hbm.at[pl.ds(off, GR2)], idx_v)
        pltpu.async_copy(r_hbm.at[idx_v], rel_v, sem).wait()
        pltpu.sync_copy(dst_hbm.at[pl.ds(off, GR2)], idx_v)
        pltpu.async_copy(q_hbm.at[idx_v], qd_v, sem).wait()

        def row(r, carry2):
            for f in range(D // L):
                sl = pl.ds(f * L, L)
                xs_v[r, sl] = xs_v[r, sl] * rel_v[r, sl]
            return carry2

        lax.fori_loop(0, GR2, row, 0)
        pltpu.sync_copy(xs_v, msg_hbm.at[pl.ds(off, GR2)])
        pltpu.sync_copy(qd_v, qd_hbm.at[pl.ds(off, GR2)])
        return carry

    lax.fori_loop(0, NCH2, chunk, 0)


def _attn_body(msg_ref, qd_ref, wk_ref, wv_ref, p_ref, sv_ref):
    msg = msg_ref[...]
    k = jnp.dot(msg, wk_ref[...], preferred_element_type=jnp.float32)
    v = jnp.dot(msg, wv_ref[...], preferred_element_type=jnp.float32)
    a = qd_ref[...] * k
    asum = jnp.dot(a, p_ref[...], preferred_element_type=jnp.float32)
    w = jax.nn.sigmoid(asum * (1.0 / math.sqrt(HD)))
    sv_ref[...] = w * v


def _attn(msg, qd, Wk, Wv, P):
    grid = (E // BE,)
    return pl.pallas_call(
        _attn_body,
        grid=grid,
        in_specs=[
            pl.BlockSpec((BE, D), lambda i: (i, 0)),
            pl.BlockSpec((BE, D), lambda i: (i, 0)),
            pl.BlockSpec((D, D), lambda i: (0, 0)),
            pl.BlockSpec((D, D), lambda i: (0, 0)),
            pl.BlockSpec((D, D), lambda i: (0, 0)),
        ],
        out_specs=pl.BlockSpec((BE, D), lambda i: (i, 0)),
        out_shape=jax.ShapeDtypeStruct((E, D), jnp.float32),
        compiler_params=pltpu.CompilerParams(
            dimension_semantics=("parallel",)),
    )(msg, qd, Wk, Wv, P)


@functools.partial(
    pl.kernel,
    out_type=(jax.ShapeDtypeStruct((N_PAD, D), jnp.float32),
              jax.ShapeDtypeStruct((N_PAD, D), jnp.float32),
              jax.ShapeDtypeStruct((N_PAD,), jnp.float32)),
    mesh=_mesh,
    scratch_types=[
        pltpu.VMEM((RW, D), jnp.float32),
        pltpu.VMEM((RW, D), jnp.float32),
        pltpu.VMEM((RW + L,), jnp.float32),
        pltpu.VMEM((CH + L,), jnp.int32),
        pltpu.VMEM((MCAP,), jnp.int32),
        pltpu.VMEM((GR4, D), jnp.float32),
        pltpu.SemaphoreType.DMA,
    ],
)
def _sc_scatter(dst_hbm, sv_hbm, s_hbm, m_hbm, deg_hbm,
                s_v, m_v, deg_v, dstc_v, midx_v, rows_v, sem):
    base = _wid() * RW
    zeroes = jnp.zeros((L,), jnp.float32)
    neginf = jnp.full((L,), -3.0e38, jnp.float32)
    izeroes = jnp.zeros((L,), jnp.int32)
    one_hot0 = jnp.where(lax.iota(jnp.int32, L) == 0, 1.0, 0.0
                         ).astype(jnp.float32)

    def init_row(i, carry):
        for f in range(D // L):
            s_v[i, pl.ds(f * L, L)] = zeroes
            m_v[i, pl.ds(f * L, L)] = neginf
        return carry

    lax.fori_loop(0, RW, init_row, 0)

    def init_deg(i, carry):
        deg_v[pl.ds(i * L, L)] = zeroes
        return carry

    lax.fori_loop(0, (RW + L) // L, init_deg, 0)

    def init_midx(i, carry):
        midx_v[pl.ds(i * L, L)] = izeroes
        return carry

    lax.fori_loop(0, MCAP // L, init_midx, 0)

    lane_iota = lax.iota(jnp.int32, L)

    def chunk(c, carry):
        pltpu.sync_copy(dst_hbm.at[pl.ds(c * CH, CH)],
                        dstc_v.at[pl.ds(0, CH)])

        def grp(i, nm):
            d = dstc_v[pl.ds(i * L, L)]
            ld = d - base
            mki = jnp.where((ld >= 0) & (ld < RW), 1, 0).astype(jnp.int32)
            egrp = c * CH + i * L
            for j in range(L):
                midx_v[pl.ds(nm, L)] = jnp.full((L,), egrp + j, jnp.int32)
                nm = nm + mki[j]
            return nm

        nm = lax.fori_loop(0, CH // L, grp, jnp.int32(0))
        ngr = (nm + GR4 - 1) // GR4

        def gather_grp(g, carry2):
            pltpu.async_copy(sv_hbm.at[midx_v.at[pl.ds(g * GR4, GR4)]],
                             rows_v, sem).wait()
            nrow = jnp.minimum(nm - g * GR4, GR4)

            def rowfn(k2, carry3):
                ev = midx_v[pl.ds(g * GR4 + k2, L)][0]
                l = dstc_v[pl.ds(ev - c * CH, L)][0] - base
                for f in range(D // L):
                    sl = pl.ds(f * L, L)
                    rv = rows_v[k2, sl]
                    s_v[l, sl] = s_v[l, sl] + rv
                    m_v[l, sl] = jnp.maximum(m_v[l, sl], rv)
                dsl = pl.ds(l, L)
                deg_v[dsl] = deg_v[dsl] + one_hot0
                return carry3

            lax.fori_loop(0, nrow, rowfn, 0)
            return carry2

        lax.fori_loop(0, ngr, gather_grp, 0)
        return carry

    lax.fori_loop(0, NCH4, chunk, 0)
    pltpu.sync_copy(s_v, s_hbm.at[pl.ds(base, RW)])
    pltpu.sync_copy(m_v, m_hbm.at[pl.ds(base, RW)])
    pltpu.sync_copy(deg_v.at[pl.ds(0, RW)], deg_hbm.at[pl.ds(base, RW)])


def _epi_body(x_ref, s_ref, m_ref, deg_ref, a_ref, b_ref, c_ref, d_ref,
              g_ref, bt_ref, o_ref):
    S = s_ref[...]
    degb = deg_ref[...]
    M = jnp.where(degb == 0.0, 0.0, m_ref[...])
    sc = 1.0 / jnp.sqrt(jnp.maximum(degb, 1.0))
    c1 = (jnp.dot(S, a_ref[...], preferred_element_type=jnp.float32)
          + jnp.dot(M, c_ref[...], preferred_element_type=jnp.float32))
    c2 = (jnp.dot(S, b_ref[...], preferred_element_type=jnp.float32)
          + jnp.dot(M, d_ref[...], preferred_element_type=jnp.float32))
    h = x_ref[...] + c1 + sc * c2
    mu = jnp.mean(h, axis=1, keepdims=True)
    var = jnp.mean((h - mu) * (h - mu), axis=1, keepdims=True)
    o_ref[...] = ((h - mu) / jnp.sqrt(var + 1e-5)) * g_ref[...] + bt_ref[...]


def _epilogue(x, S, M, deg, A, B, C, Dm, gamma, beta):
    grid = (N_PAD // BN,)
    blk = lambda i: (i, 0)
    zero = lambda i: (0, 0)
    return pl.pallas_call(
        _epi_body,
        grid=grid,
        in_specs=[
            pl.BlockSpec((BN, D), blk),
            pl.BlockSpec((BN, D), blk),
            pl.BlockSpec((BN, D), blk),
            pl.BlockSpec((BN, D), blk),
            pl.BlockSpec((D, D), zero),
            pl.BlockSpec((D, D), zero),
            pl.BlockSpec((D, D), zero),
            pl.BlockSpec((D, D), zero),
            pl.BlockSpec((1, D), zero),
            pl.BlockSpec((1, D), zero),
        ],
        out_specs=pl.BlockSpec((BN, D), blk),
        out_shape=jax.ShapeDtypeStruct((N_PAD, D), jnp.float32),
        compiler_params=pltpu.CompilerParams(
            dimension_semantics=("parallel",)),
    )(x, S, M, deg, A, B, C, Dm, gamma, beta)


def kernel(x, edge_index, edge_type, Wq, Wk, Wv, R, W_o, ln_gamma, ln_beta):
    src = edge_index[0].astype(jnp.int32)
    dst = edge_index[1].astype(jnp.int32)
    et = edge_type.astype(jnp.int32)

    A = W_o[0:D] + W_o[D:3 * D:2]
    B = W_o[D + 1:3 * D:2]
    C = W_o[3 * D:5 * D:2]
    Dm = W_o[3 * D + 1:5 * D:2]
    P = jnp.kron(jnp.eye(H, dtype=jnp.float32),
                 jnp.ones((HD, HD), dtype=jnp.float32))

    q = _q_proj(x, Wq)
    msg, qd = _sc_gather(x, q, R, src, dst, et)
    sv = _attn(msg, qd, Wk, Wv, P)
    S, M, deg = _sc_scatter(dst, sv)
    x_pad = jnp.pad(x, ((0, N_PAD - N), (0, 0)))
    degb = jnp.broadcast_to(deg[:, None], (N_PAD, D))
    out = _epilogue(x_pad, S, M, degb, A, B, C, Dm,
                    ln_gamma.reshape(1, D), ln_beta.reshape(1, D))
    return out[:N]


# speedup vs baseline: 3.9810x; 2.6840x over previous
import functools
import math

import jax
import jax.numpy as jnp
from jax import lax
from jax.experimental import pallas as pl
from jax.experimental.pallas import tpu as pltpu
from jax.experimental.pallas import tpu_sc as plsc

N = 10000
E = 320000
D = 128
H = 4
HD = D // H
NR = 64

NC = 2
NS = 16
NW = NC * NS
L = 16

N_PAD = 10240
RW = N_PAD // NW

EW = E // NW
GR2 = 200
NCH2 = EW // GR2

CH = 3200
NCH4 = E // CH
GR4 = 128
MCAP = 3264

BE = 3200
BN = 2048

_mesh = plsc.VectorSubcoreMesh(core_axis_name="c", subcore_axis_name="s")


def _wid():
    return lax.axis_index("s") * NC + lax.axis_index("c")


def _q_body(x_ref, wq_ref, q_ref):
    q_ref[...] = jnp.dot(x_ref[...], wq_ref[...],
                         preferred_element_type=jnp.float32)


def _q_proj(x, Wq):
    return pl.pallas_call(
        _q_body,
        out_shape=jax.ShapeDtypeStruct((N, D), jnp.float32),
    )(x, Wq)


@functools.partial(
    pl.kernel,
    out_type=(jax.ShapeDtypeStruct((E, D), jnp.float32),
              jax.ShapeDtypeStruct((E, D), jnp.float32)),
    mesh=_mesh,
    scratch_types=[
        pltpu.VMEM((GR2,), jnp.int32),
        pltpu.VMEM((GR2,), jnp.int32),
        pltpu.VMEM((GR2,), jnp.int32),
        pltpu.VMEM((GR2, D), jnp.float32),
        pltpu.VMEM((GR2, D), jnp.float32),
        pltpu.VMEM((GR2, D), jnp.float32),
        pltpu.SemaphoreType.DMA,
        pltpu.SemaphoreType.DMA,
        pltpu.SemaphoreType.DMA,
    ],
)
def _sc_gather(x_hbm, q_hbm, r_hbm, src_hbm, dst_hbm, et_hbm,
               msg_hbm, qd_hbm, isrc_v, iet_v, idst_v,
               xs_v, rel_v, qd_v, sem_a, sem_b, sem_c):
    ebase = _wid() * EW

    def chunk(c, carry):
        off = ebase + c * GR2
        hs = pltpu.async_copy(src_hbm.at[pl.ds(off, GR2)], isrc_v, sem_a)
        he = pltpu.async_copy(et_hbm.at[pl.ds(off, GR2)], iet_v, sem_b)
        hd = pltpu.async_copy(dst_hbm.at[pl.ds(off, GR2)], idst_v, sem_c)
        hs.wait()
        g1 = pltpu.async_copy(x_hbm.at[isrc_v], xs_v, sem_a)
        he.wait()
        g2 = pltpu.async_copy(r_hbm.at[iet_v], rel_v, sem_b)
        hd.wait()
        g3 = pltpu.async_copy(q_hbm.at[idst_v], qd_v, sem_c)
        g1.wait()
        g2.wait()

        def row(r, carry2):
            for f in range(D // L):
                sl = pl.ds(f * L, L)
                xs_v[r, sl] = xs_v[r, sl] * rel_v[r, sl]
            return carry2

        lax.fori_loop(0, GR2, row, 0)
        pltpu.sync_copy(xs_v, msg_hbm.at[pl.ds(off, GR2)])
        g3.wait()
        pltpu.sync_copy(qd_v, qd_hbm.at[pl.ds(off, GR2)])
        return carry

    lax.fori_loop(0, NCH2, chunk, 0)


def _attn_body(msg_ref, qd_ref, wk_ref, wv_ref, p_ref, sv_ref):
    msg = msg_ref[...]
    k = jnp.dot(msg, wk_ref[...], preferred_element_type=jnp.float32)
    v = jnp.dot(msg, wv_ref[...], preferred_element_type=jnp.float32)
    a = qd_ref[...] * k
    asum = jnp.dot(a, p_ref[...], preferred_element_type=jnp.float32)
    w = jax.nn.sigmoid(asum * (1.0 / math.sqrt(HD)))
    sv_ref[...] = w * v


def _attn(msg, qd, Wk, Wv, P):
    grid = (E // BE,)
    return pl.pallas_call(
        _attn_body,
        grid=grid,
        in_specs=[
            pl.BlockSpec((BE, D), lambda i: (i, 0)),
            pl.BlockSpec((BE, D), lambda i: (i, 0)),
            pl.BlockSpec((D, D), lambda i: (0, 0)),
            pl.BlockSpec((D, D), lambda i: (0, 0)),
            pl.BlockSpec((D, D), lambda i: (0, 0)),
        ],
        out_specs=pl.BlockSpec((BE, D), lambda i: (i, 0)),
        out_shape=jax.ShapeDtypeStruct((E, D), jnp.float32),
        compiler_params=pltpu.CompilerParams(
            dimension_semantics=("parallel",)),
    )(msg, qd, Wk, Wv, P)


@functools.partial(
    pl.kernel,
    out_type=(jax.ShapeDtypeStruct((N_PAD, D), jnp.float32),
              jax.ShapeDtypeStruct((N_PAD, D), jnp.float32),
              jax.ShapeDtypeStruct((N_PAD,), jnp.float32)),
    mesh=_mesh,
    scratch_types=[
        pltpu.VMEM((RW, D), jnp.float32),
        pltpu.VMEM((RW, D), jnp.float32),
        pltpu.VMEM((RW + L,), jnp.float32),
        pltpu.VMEM((CH + L,), jnp.int32),
        pltpu.VMEM((CH + L,), jnp.int32),
        pltpu.VMEM((MCAP,), jnp.int32),
        pltpu.VMEM((MCAP,), jnp.int32),
        pltpu.VMEM((GR4 + L,), jnp.int32),
        pltpu.VMEM((GR4 + L,), jnp.int32),
        pltpu.VMEM((GR4, D), jnp.float32),
        pltpu.VMEM((GR4, D), jnp.float32),
        pltpu.SemaphoreType.DMA,
        pltpu.SemaphoreType.DMA,
        pltpu.SemaphoreType.DMA,
        pltpu.SemaphoreType.DMA,
        pltpu.SemaphoreType.DMA,
        pltpu.SemaphoreType.DMA,
    ],
)
def _sc_scatter(dst_hbm, sv_hbm, s_hbm, m_hbm, deg_hbm,
                s_v, m_v, deg_v, dstc_a, dstc_b, midx_a, midx_b,
                dval_a, dval_b, rows_a, rows_b,
                sem_la, sem_lb, sem_ga, sem_gb, sem_da, sem_db):
    base = _wid() * RW
    zeroes = jnp.zeros((L,), jnp.float32)
    neginf = jnp.full((L,), -3.0e38, jnp.float32)
    izeroes = jnp.zeros((L,), jnp.int32)
    one_hot0 = jnp.where(lax.iota(jnp.int32, L) == 0, 1.0, 0.0
                         ).astype(jnp.float32)

    def init_row(i, carry):
        for f in range(D // L):
            s_v[i, pl.ds(f * L, L)] = zeroes
            m_v[i, pl.ds(f * L, L)] = neginf
        return carry

    lax.fori_loop(0, RW, init_row, 0)

    def init_deg(i, carry):
        deg_v[pl.ds(i * L, L)] = zeroes
        return carry

    lax.fori_loop(0, (RW + L) // L, init_deg, 0)

    def init_midx(i, carry):
        midx_a[pl.ds(i * L, L)] = izeroes
        midx_b[pl.ds(i * L, L)] = izeroes
        return carry

    lax.fori_loop(0, MCAP // L, init_midx, 0)

    def compact(c, dstc_v, midx_v):
        def grp(i, nm):
            d = dstc_v[pl.ds(i * L, L)]
            ld = d - base
            mki = jnp.where((ld >= 0) & (ld < RW), 1, 0).astype(jnp.int32)
            egrp = c * CH + i * L
            for j in range(L):
                midx_v[pl.ds(nm, L)] = jnp.full((L,), egrp + j, jnp.int32)
                nm = nm + mki[j]
            return nm

        return lax.fori_loop(0, CH // L, grp, jnp.int32(0))

    def start_gather(g, midx_v, rows_v, dval_v, sem_g, sem_d):
        hr = pltpu.async_copy(sv_hbm.at[midx_v.at[pl.ds(g * GR4, GR4)]],
                              rows_v, sem_g)
        hd = pltpu.async_copy(dst_hbm.at[midx_v.at[pl.ds(g * GR4, GR4)]],
                              dval_v.at[pl.ds(0, GR4)], sem_d)
        return hr, hd

    def process_rows(nrow, rows_v, dval_v):
        def rowfn(k2, carry3):
            l = dval_v[pl.ds(k2, L)][0] - base
            for f in range(D // L):
                sl = pl.ds(f * L, L)
                rv = rows_v[k2, sl]
                s_v[l, sl] = s_v[l, sl] + rv
                m_v[l, sl] = jnp.maximum(m_v[l, sl], rv)
            dsl = pl.ds(l, L)
            deg_v[dsl] = deg_v[dsl] + one_hot0
            return carry3

        lax.fori_loop(0, nrow, rowfn, 0)

    def process_chunk(nm, midx_v, rows_v, dval_v, sem_g, sem_d):
        process_rows(jnp.minimum(nm, GR4), rows_v, dval_v)
        ngr = (nm + GR4 - 1) // GR4

        def extra(gi, carry2):
            hr, hd = start_gather(gi, midx_v, rows_v, dval_v, sem_g, sem_d)
            hr.wait()
            hd.wait()
            process_rows(jnp.minimum(nm - gi * GR4, GR4), rows_v, dval_v)
            return carry2

        lax.fori_loop(1, jnp.maximum(ngr, 1), extra, 0)

    def pair(p, carry):
        ca = 2 * p
        cb = 2 * p + 1
        hla = pltpu.async_copy(dst_hbm.at[pl.ds(ca * CH, CH)],
                               dstc_a.at[pl.ds(0, CH)], sem_la)
        hlb = pltpu.async_copy(dst_hbm.at[pl.ds(cb * CH, CH)],
                               dstc_b.at[pl.ds(0, CH)], sem_lb)
        hla.wait()
        nma = compact(ca, dstc_a, midx_a)
        hra, hda = start_gather(0, midx_a, rows_a, dval_a, sem_ga, sem_da)
        hlb.wait()
        nmb = compact(cb, dstc_b, midx_b)
        hrb, hdb = start_gather(0, midx_b, rows_b, dval_b, sem_gb, sem_db)
        hra.wait()
        hda.wait()
        process_chunk(nma, midx_a, rows_a, dval_a, sem_ga, sem_da)
        hrb.wait()
        hdb.wait()
        process_chunk(nmb, midx_b, rows_b, dval_b, sem_gb, sem_db)
        return carry

    lax.fori_loop(0, NCH4 // 2, pair, 0)
    pltpu.sync_copy(s_v, s_hbm.at[pl.ds(base, RW)])
    pltpu.sync_copy(m_v, m_hbm.at[pl.ds(base, RW)])
    pltpu.sync_copy(deg_v.at[pl.ds(0, RW)], deg_hbm.at[pl.ds(base, RW)])


def _epi_body(x_ref, s_ref, m_ref, deg_ref, a_ref, b_ref, c_ref, d_ref,
              g_ref, bt_ref, o_ref):
    S = s_ref[...]
    degb = deg_ref[...]
    M = jnp.where(degb == 0.0, 0.0, m_ref[...])
    sc = 1.0 / jnp.sqrt(jnp.maximum(degb, 1.0))
    c1 = (jnp.dot(S, a_ref[...], preferred_element_type=jnp.float32)
          + jnp.dot(M, c_ref[...], preferred_element_type=jnp.float32))
    c2 = (jnp.dot(S, b_ref[...], preferred_element_type=jnp.float32)
          + jnp.dot(M, d_ref[...], preferred_element_type=jnp.float32))
    h = x_ref[...] + c1 + sc * c2
    mu = jnp.mean(h, axis=1, keepdims=True)
    var = jnp.mean((h - mu) * (h - mu), axis=1, keepdims=True)
    o_ref[...] = ((h - mu) / jnp.sqrt(var + 1e-5)) * g_ref[...] + bt_ref[...]


def _epilogue(x, S, M, deg, A, B, C, Dm, gamma, beta):
    grid = (N_PAD // BN,)
    blk = lambda i: (i, 0)
    zero = lambda i: (0, 0)
    return pl.pallas_call(
        _epi_body,
        grid=grid,
        in_specs=[
            pl.BlockSpec((BN, D), blk),
            pl.BlockSpec((BN, D), blk),
            pl.BlockSpec((BN, D), blk),
            pl.BlockSpec((BN, D), blk),
            pl.BlockSpec((D, D), zero),
            pl.BlockSpec((D, D), zero),
            pl.BlockSpec((D, D), zero),
            pl.BlockSpec((D, D), zero),
            pl.BlockSpec((1, D), zero),
            pl.BlockSpec((1, D), zero),
        ],
        out_specs=pl.BlockSpec((BN, D), blk),
        out_shape=jax.ShapeDtypeStruct((N_PAD, D), jnp.float32),
        compiler_params=pltpu.CompilerParams(
            dimension_semantics=("parallel",)),
    )(x, S, M, deg, A, B, C, Dm, gamma, beta)


def kernel(x, edge_index, edge_type, Wq, Wk, Wv, R, W_o, ln_gamma, ln_beta):
    src = edge_index[0].astype(jnp.int32)
    dst = edge_index[1].astype(jnp.int32)
    et = edge_type.astype(jnp.int32)

    A = W_o[0:D] + W_o[D:3 * D:2]
    B = W_o[D + 1:3 * D:2]
    C = W_o[3 * D:5 * D:2]
    Dm = W_o[3 * D + 1:5 * D:2]
    P = jnp.kron(jnp.eye(H, dtype=jnp.float32),
                 jnp.ones((HD, HD), dtype=jnp.float32))

    q = _q_proj(x, Wq)
    msg, qd = _sc_gather(x, q, R, src, dst, et)
    sv = _attn(msg, qd, Wk, Wv, P)
    S, M, deg = _sc_scatter(dst, sv)
    x_pad = jnp.pad(x, ((0, N_PAD - N), (0, 0)))
    degb = jnp.broadcast_to(deg[:, None], (N_PAD, D))
    out = _epilogue(x_pad, S, M, degb, A, B, C, Dm,
                    ln_gamma.reshape(1, D), ln_beta.reshape(1, D))
    return out[:N]
